# baseline (speedup 1.0000x reference)
import functools

import jax
import jax.numpy as jnp
from jax import lax
from jax.experimental import pallas as pl
from jax.experimental.pallas import tpu as pltpu
from jax.experimental.pallas import tpu_sc as plsc

D = 768
CHUNK = 64
TC_BLK = 512


def _sc_gather(xf, tok_table, *, n_tok):
    info = plsc.get_sparse_core_info()
    nw = info.num_cores * info.num_subcores
    per_w = n_tok // nw
    n_chunks = per_w // CHUNK

    mesh = plsc.VectorSubcoreMesh(core_axis_name="c", subcore_axis_name="s")

    @functools.partial(
        pl.kernel,
        mesh=mesh,
        out_type=jax.ShapeDtypeStruct((n_tok, D), jnp.float32),
        compiler_params=pltpu.CompilerParams(needs_layout_passes=False),
        scratch_types=[
            pltpu.VMEM((n_chunks, CHUNK), jnp.int32),
            pltpu.VMEM((CHUNK, D), jnp.float32),
            pltpu.VMEM((CHUNK, D), jnp.float32),
            pltpu.SemaphoreType.DMA,
            pltpu.SemaphoreType.DMA,
            pltpu.SemaphoreType.DMA,
        ],
    )
    def k(x_hbm, tok_hbm, out_hbm, idx_v, buf0, buf1, gsem, osem0, osem1):
        wid = lax.axis_index("s") * info.num_cores + lax.axis_index("c")
        base = wid * per_w
        pltpu.sync_copy(x_hbm.at[wid], idx_v)

        bufs = (buf0, buf1)
        osems = (osem0, osem1)
        pltpu.async_copy(tok_hbm.at[idx_v.at[0]], buf0, gsem).wait()
        for ci in range(1, n_chunks + 1):
            cur = (ci - 1) % 2
            nxt = ci % 2
            gn = None
            if ci < n_chunks:
                gn = pltpu.async_copy(tok_hbm.at[idx_v.at[ci]], bufs[nxt], gsem)
            out = pltpu.async_copy(
                bufs[cur], out_hbm.at[pl.ds(base + (ci - 1) * CHUNK, CHUNK)],
                osems[cur])
            if gn is not None:
                gn.wait()
            out.wait()

    return k(xf.reshape(nw, n_chunks, CHUNK), tok_table)


def _tc_add_ln(rows, pos_table, gamma2d, beta2d, *, seq_len, n_out_tok,
               base_blk, prev=None):
    n_blk = rows.shape[0] // TC_BLK
    pos_blocks = seq_len // TC_BLK
    n_batch = n_blk // pos_blocks

    def body(r_ref, p_ref, g_ref, b_ref, *rest):
        o_ref = rest[-1]
        h = r_ref[...] + p_ref[...]
        mean = jnp.mean(h, axis=-1, keepdims=True)
        c = h - mean
        var = jnp.mean(c * c, axis=-1, keepdims=True)
        inv = lax.rsqrt(var + 1e-5)
        o_ref[...] = c * inv * g_ref[...] + b_ref[...]

    in_specs = [
        pl.BlockSpec((TC_BLK, D), lambda i, j: (j * pos_blocks + i, 0)),
        pl.BlockSpec((TC_BLK, D), lambda i, j: (i, 0)),
        pl.BlockSpec((1, D), lambda i, j: (0, 0)),
        pl.BlockSpec((1, D), lambda i, j: (0, 0)),
    ]
    args = [rows, pos_table, gamma2d, beta2d]
    kwargs = {}
    if prev is not None:
        in_specs.append(pl.BlockSpec((8, D), lambda i, j: (0, 0)))
        args.append(prev)
        kwargs["input_output_aliases"] = {4: 0}
    return pl.pallas_call(
        body,
        grid=(pos_blocks, n_batch),
        in_specs=in_specs,
        out_specs=pl.BlockSpec(
            (TC_BLK, D), lambda i, j: (base_blk + j * pos_blocks + i, 0)),
        out_shape=jax.ShapeDtypeStruct((n_out_tok, D), jnp.float32),
        **kwargs,
    )(*args)


N_SLICES = 4


def kernel(x, tok_table, pos_table, gamma, beta):
    b, s = x.shape
    n_tok = b * s
    xf = x.reshape(n_tok).astype(jnp.int32)
    g2 = gamma.reshape(1, D)
    b2 = beta.reshape(1, D)
    per_slice = n_tok // N_SLICES
    rows = [
        _sc_gather(lax.slice(xf, (k * per_slice,), ((k + 1) * per_slice,)),
                   tok_table, n_tok=per_slice)
        for k in range(N_SLICES)
    ]
    out = None
    for k in range(N_SLICES):
        out = _tc_add_ln(rows[k], pos_table, g2, b2, seq_len=s,
                         n_out_tok=n_tok, base_blk=k * (per_slice // TC_BLK),
                         prev=out)
    return out.reshape(b, s, D)


# speedup vs baseline: 1.1006x; 1.1006x over previous
import functools

import jax
import jax.numpy as jnp
from jax import lax
from jax.experimental import pallas as pl
from jax.experimental.pallas import tpu as pltpu
from jax.experimental.pallas import tpu_sc as plsc

D = 768
CHUNK = 64
TC_BLK = 1024


def _sc_gather(xf, tok_table, *, n_tok):
    info = plsc.get_sparse_core_info()
    nw = info.num_cores * info.num_subcores
    per_w = n_tok // nw
    n_chunks = per_w // CHUNK

    mesh = plsc.VectorSubcoreMesh(core_axis_name="c", subcore_axis_name="s")

    @functools.partial(
        pl.kernel,
        mesh=mesh,
        out_type=jax.ShapeDtypeStruct((n_tok, D), jnp.float32),
        compiler_params=pltpu.CompilerParams(needs_layout_passes=False),
        scratch_types=[
            pltpu.VMEM((n_chunks, CHUNK), jnp.int32),
            pltpu.VMEM((CHUNK, D), jnp.float32),
            pltpu.VMEM((CHUNK, D), jnp.float32),
            pltpu.SemaphoreType.DMA,
            pltpu.SemaphoreType.DMA,
            pltpu.SemaphoreType.DMA,
        ],
    )
    def k(x_hbm, tok_hbm, out_hbm, idx_v, buf0, buf1, gsem, osem0, osem1):
        wid = lax.axis_index("s") * info.num_cores + lax.axis_index("c")
        base = wid * per_w
        pltpu.sync_copy(x_hbm.at[wid], idx_v)

        bufs = (buf0, buf1)
        osems = (osem0, osem1)
        pltpu.async_copy(tok_hbm.at[idx_v.at[0]], buf0, gsem).wait()
        for ci in range(1, n_chunks + 1):
            cur = (ci - 1) % 2
            nxt = ci % 2
            gn = None
            if ci < n_chunks:
                gn = pltpu.async_copy(tok_hbm.at[idx_v.at[ci]], bufs[nxt], gsem)
            out = pltpu.async_copy(
                bufs[cur], out_hbm.at[pl.ds(base + (ci - 1) * CHUNK, CHUNK)],
                osems[cur])
            if gn is not None:
                gn.wait()
            out.wait()

    return k(xf.reshape(nw, n_chunks, CHUNK), tok_table)


def _tc_add_ln(rows, pos_table, gamma2d, beta2d, *, seq_len, n_out_tok,
               base_blk, prev=None):
    n_blk = rows.shape[0] // TC_BLK
    pos_blocks = seq_len // TC_BLK
    n_batch = n_blk // pos_blocks

    def body(r_ref, p_ref, g_ref, b_ref, *rest):
        o_ref = rest[-1]
        h = r_ref[...] + p_ref[...]
        mean = jnp.mean(h, axis=-1, keepdims=True)
        c = h - mean
        var = jnp.mean(c * c, axis=-1, keepdims=True)
        inv = lax.rsqrt(var + 1e-5)
        o_ref[...] = c * inv * g_ref[...] + b_ref[...]

    in_specs = [
        pl.BlockSpec((TC_BLK, D), lambda i, j: (j * pos_blocks + i, 0)),
        pl.BlockSpec((TC_BLK, D), lambda i, j: (i, 0)),
        pl.BlockSpec((1, D), lambda i, j: (0, 0)),
        pl.BlockSpec((1, D), lambda i, j: (0, 0)),
    ]
    args = [rows, pos_table, gamma2d, beta2d]
    kwargs = {}
    if prev is not None:
        in_specs.append(pl.BlockSpec((8, D), lambda i, j: (0, 0)))
        args.append(prev)
        kwargs["input_output_aliases"] = {4: 0}
    return pl.pallas_call(
        body,
        grid=(pos_blocks, n_batch),
        in_specs=in_specs,
        out_specs=pl.BlockSpec(
            (TC_BLK, D), lambda i, j: (base_blk + j * pos_blocks + i, 0)),
        out_shape=jax.ShapeDtypeStruct((n_out_tok, D), jnp.float32),
        **kwargs,
    )(*args)


N_SLICES = 2


def kernel(x, tok_table, pos_table, gamma, beta):
    b, s = x.shape
    n_tok = b * s
    xf = x.reshape(n_tok).astype(jnp.int32)
    g2 = gamma.reshape(1, D)
    b2 = beta.reshape(1, D)
    per_slice = n_tok // N_SLICES
    rows = [
        _sc_gather(lax.slice(xf, (k * per_slice,), ((k + 1) * per_slice,)),
                   tok_table, n_tok=per_slice)
        for k in range(N_SLICES)
    ]
    out = None
    for k in range(N_SLICES):
        out = _tc_add_ln(rows[k], pos_table, g2, b2, seq_len=s,
                         n_out_tok=n_tok, base_blk=k * (per_slice // TC_BLK),
                         prev=out)
    return out.reshape(b, s, D)


# speedup vs baseline: 1.1519x; 1.0466x over previous
import functools

import jax
import jax.numpy as jnp
from jax import lax
from jax.experimental import pallas as pl
from jax.experimental.pallas import tpu as pltpu
from jax.experimental.pallas import tpu_sc as plsc

D = 768
CHUNK = 64
TC_BLK = 2048


def _sc_gather(xf, tok_table, *, n_tok):
    info = plsc.get_sparse_core_info()
    nw = info.num_cores * info.num_subcores
    per_w = n_tok // nw
    n_chunks = per_w // CHUNK

    mesh = plsc.VectorSubcoreMesh(core_axis_name="c", subcore_axis_name="s")

    @functools.partial(
        pl.kernel,
        mesh=mesh,
        out_type=jax.ShapeDtypeStruct((n_tok, D), jnp.float32),
        compiler_params=pltpu.CompilerParams(needs_layout_passes=False),
        scratch_types=[
            pltpu.VMEM((n_chunks, CHUNK), jnp.int32),
            pltpu.VMEM((CHUNK, D), jnp.float32),
            pltpu.VMEM((CHUNK, D), jnp.float32),
            pltpu.SemaphoreType.DMA,
            pltpu.SemaphoreType.DMA,
            pltpu.SemaphoreType.DMA,
        ],
    )
    def k(x_hbm, tok_hbm, out_hbm, idx_v, buf0, buf1, gsem, osem0, osem1):
        wid = lax.axis_index("s") * info.num_cores + lax.axis_index("c")
        base = wid * per_w
        pltpu.sync_copy(x_hbm.at[wid], idx_v)

        bufs = (buf0, buf1)
        osems = (osem0, osem1)
        pltpu.async_copy(tok_hbm.at[idx_v.at[0]], buf0, gsem).wait()
        for ci in range(1, n_chunks + 1):
            cur = (ci - 1) % 2
            nxt = ci % 2
            gn = None
            if ci < n_chunks:
                gn = pltpu.async_copy(tok_hbm.at[idx_v.at[ci]], bufs[nxt], gsem)
            out = pltpu.async_copy(
                bufs[cur], out_hbm.at[pl.ds(base + (ci - 1) * CHUNK, CHUNK)],
                osems[cur])
            if gn is not None:
                gn.wait()
            out.wait()

    return k(xf.reshape(nw, n_chunks, CHUNK), tok_table)


def _tc_add_ln(rows, pos_table, gamma2d, beta2d, *, seq_len, n_out_tok,
               base_blk, prev=None):
    n_blk = rows.shape[0] // TC_BLK
    pos_blocks = seq_len // TC_BLK
    n_batch = n_blk // pos_blocks

    def body(r_ref, p_ref, g_ref, b_ref, *rest):
        o_ref = rest[-1]
        h = r_ref[...] + p_ref[...]
        mean = jnp.mean(h, axis=-1, keepdims=True)
        c = h - mean
        var = jnp.mean(c * c, axis=-1, keepdims=True)
        inv = lax.rsqrt(var + 1e-5)
        o_ref[...] = c * inv * g_ref[...] + b_ref[...]

    in_specs = [
        pl.BlockSpec((TC_BLK, D), lambda i, j: (j * pos_blocks + i, 0)),
        pl.BlockSpec((TC_BLK, D), lambda i, j: (i, 0)),
        pl.BlockSpec((1, D), lambda i, j: (0, 0)),
        pl.BlockSpec((1, D), lambda i, j: (0, 0)),
    ]
    args = [rows, pos_table, gamma2d, beta2d]
    kwargs = {}
    if prev is not None:
        in_specs.append(pl.BlockSpec((8, D), lambda i, j: (0, 0)))
        args.append(prev)
        kwargs["input_output_aliases"] = {4: 0}
    return pl.pallas_call(
        body,
        grid=(pos_blocks, n_batch),
        in_specs=in_specs,
        out_specs=pl.BlockSpec(
            (TC_BLK, D), lambda i, j: (base_blk + j * pos_blocks + i, 0)),
        out_shape=jax.ShapeDtypeStruct((n_out_tok, D), jnp.float32),
        **kwargs,
    )(*args)


N_SLICES = 2


def kernel(x, tok_table, pos_table, gamma, beta):
    b, s = x.shape
    n_tok = b * s
    xf = x.reshape(n_tok).astype(jnp.int32)
    g2 = gamma.reshape(1, D)
    b2 = beta.reshape(1, D)
    per_slice = n_tok // N_SLICES
    rows = [
        _sc_gather(lax.slice(xf, (k * per_slice,), ((k + 1) * per_slice,)),
                   tok_table, n_tok=per_slice)
        for k in range(N_SLICES)
    ]
    out = None
    for k in range(N_SLICES):
        out = _tc_add_ln(rows[k], pos_table, g2, b2, seq_len=s,
                         n_out_tok=n_tok, base_blk=k * (per_slice // TC_BLK),
                         prev=out)
    return out.reshape(b, s, D)
